# baseline (speedup 1.0000x reference)
import jax
import jax.numpy as jnp
from jax.experimental import pallas as pl
from jax.experimental.pallas import tpu as pltpu

_TILE = 512
_HALF = 1024


def _gate_kernel(x_ref, gw_ref, idx_ref, rw_ref):
    x = x_ref[...]
    logits = jax.lax.dot_general(
        x, gw_ref[...], (((1,), (1,)), ((), ())),
        preferred_element_type=jnp.float32)
    s = jnp.sum(logits, axis=0, keepdims=True)
    e_iota = jax.lax.broadcasted_iota(jnp.int32, s.shape, 1)
    i0 = jnp.argmax(s, axis=1)[0]
    s_masked = jnp.where(e_iota == i0, -jnp.inf, s)
    i1 = jnp.argmax(s_masked, axis=1)[0]

    l0 = jnp.sum(jnp.where(e_iota == i0, logits, 0.0), axis=1, keepdims=True)
    l1 = jnp.sum(jnp.where(e_iota == i1, logits, 0.0), axis=1, keepdims=True)
    m = jnp.maximum(l0, l1)
    e0 = jnp.exp(l0 - m)
    e1 = jnp.exp(l1 - m)
    denom = e0 + e1
    w0 = e0 / denom
    w1 = e1 / denom

    k_iota = jax.lax.broadcasted_iota(jnp.int32, (1, 2), 1)
    idx_ref[...] = jnp.where(k_iota == 0, i0, i1).astype(jnp.int32)
    rw_ref[...] = jnp.concatenate([w0, w1], axis=1)


def _expert_kernel(idx_ref, x_ref, w0a_ref, w0b_ref, w1a_ref, w1b_ref,
                   b0_ref, b1_ref, rw_ref, out_ref):
    x = x_ref[...]
    xa = x[:, :_HALF]
    xb = x[:, _HALF:]
    dn = (((1,), (1,)), ((), ()))
    part0 = (jax.lax.dot_general(xa, w0a_ref[0], dn,
                                 preferred_element_type=jnp.float32) +
             jax.lax.dot_general(xb, w0b_ref[0], dn,
                                 preferred_element_type=jnp.float32))
    part1 = (jax.lax.dot_general(xa, w1a_ref[0], dn,
                                 preferred_element_type=jnp.float32) +
             jax.lax.dot_general(xb, w1b_ref[0], dn,
                                 preferred_element_type=jnp.float32))
    rw = rw_ref[...]
    w0 = rw[:, 0:1]
    w1 = rw[:, 1:2]
    j = pl.program_id(0)
    b0 = b0_ref[0, j, :].reshape(1, -1)
    b1 = b1_ref[0, j, :].reshape(1, -1)
    out_ref[...] = w0 * (part0 + b0) + w1 * (part1 + b1)


@jax.jit
def kernel(hidden_states, gate_w, expert_w, expert_b):
    B, S, d = hidden_states.shape
    T = B * S
    x = hidden_states.reshape(T, d)

    idx2d, rw = pl.pallas_call(
        _gate_kernel,
        out_shape=(
            jax.ShapeDtypeStruct((1, 2), jnp.int32),
            jax.ShapeDtypeStruct((T, 2), jnp.float32),
        ),
    )(x, gate_w)
    idx = idx2d.reshape(2)

    n_tiles = d // _TILE
    b3 = expert_b.reshape(expert_b.shape[0], n_tiles, _TILE)
    wspec_a = lambda k: pl.BlockSpec(
        (1, _TILE, _HALF), lambda j, idx_ref, k=k: (idx_ref[k], j, 0))
    wspec_b = lambda k: pl.BlockSpec(
        (1, _TILE, _HALF), lambda j, idx_ref, k=k: (idx_ref[k], j, 1))
    out = pl.pallas_call(
        _expert_kernel,
        grid_spec=pltpu.PrefetchScalarGridSpec(
            num_scalar_prefetch=1,
            grid=(n_tiles,),
            in_specs=[
                pl.BlockSpec((T, d), lambda j, idx_ref: (0, 0)),
                wspec_a(0),
                wspec_b(0),
                wspec_a(1),
                wspec_b(1),
                pl.BlockSpec((1, n_tiles, _TILE),
                             lambda j, idx_ref: (idx_ref[0], 0, 0)),
                pl.BlockSpec((1, n_tiles, _TILE),
                             lambda j, idx_ref: (idx_ref[1], 0, 0)),
                pl.BlockSpec((T, 2), lambda j, idx_ref: (0, 0)),
            ],
            out_specs=pl.BlockSpec((T, _TILE), lambda j, idx_ref: (0, j)),
        ),
        out_shape=jax.ShapeDtypeStruct((T, d), jnp.float32),
        compiler_params=pltpu.CompilerParams(
            dimension_semantics=("parallel",)),
    )(idx, x, expert_w, expert_w, expert_w, expert_w, b3, b3, rw)

    return out.reshape(B, S, d)


# speedup vs baseline: 1.2452x; 1.2452x over previous
import jax
import jax.numpy as jnp
from jax.experimental import pallas as pl
from jax.experimental.pallas import tpu as pltpu

_CH = 256
_D = 2048
_NCH = _D // _CH


def _moe_kernel(x_ref, gw_ref, b_ref, w_hbm, out_ref, wbuf, sems):
    x = x_ref[...]
    logits = jax.lax.dot_general(
        x, gw_ref[...], (((1,), (1,)), ((), ())),
        preferred_element_type=jnp.float32)
    s = jnp.sum(logits, axis=0, keepdims=True)
    e_iota = jax.lax.broadcasted_iota(jnp.int32, s.shape, 1)
    i0 = jnp.argmax(s, axis=1)[0]
    s_masked = jnp.where(e_iota == i0, -jnp.inf, s)
    i1 = jnp.argmax(s_masked, axis=1)[0]

    def copy(slot, idx, c):
        return pltpu.make_async_copy(
            w_hbm.at[idx, pl.ds(c * _CH, _CH), :],
            wbuf.at[slot, c],
            sems.at[slot, c],
        )

    def issue(c, _):
        copy(0, i0, c).start()
        copy(1, i1, c).start()
        return 0

    jax.lax.fori_loop(0, _NCH, issue, 0, unroll=True)

    l0 = jnp.sum(jnp.where(e_iota == i0, logits, 0.0), axis=1, keepdims=True)
    l1 = jnp.sum(jnp.where(e_iota == i1, logits, 0.0), axis=1, keepdims=True)
    m = jnp.maximum(l0, l1)
    e0 = jnp.exp(l0 - m)
    e1 = jnp.exp(l1 - m)
    denom = e0 + e1
    w0 = e0 / denom
    w1 = e1 / denom

    b = b_ref[...]
    row_iota = jax.lax.broadcasted_iota(jnp.int32, b.shape, 0)
    b0 = jnp.sum(jnp.where(row_iota == i0, b, 0.0), axis=0, keepdims=True)
    b1 = jnp.sum(jnp.where(row_iota == i1, b, 0.0), axis=0, keepdims=True)
    out_ref[...] = w0 * b0 + w1 * b1

    dn = (((1,), (1,)), ((), ()))

    def compute(c, _):
        copy(0, i0, c).wait()
        copy(1, i1, c).wait()
        part0 = jax.lax.dot_general(x, wbuf[0, c], dn,
                                    preferred_element_type=jnp.float32)
        part1 = jax.lax.dot_general(x, wbuf[1, c], dn,
                                    preferred_element_type=jnp.float32)
        sl = pl.ds(c * _CH, _CH)
        out_ref[:, sl] = out_ref[:, sl] + (w0 * part0 + w1 * part1)
        return 0

    jax.lax.fori_loop(0, _NCH, compute, 0, unroll=True)


@jax.jit
def kernel(hidden_states, gate_w, expert_w, expert_b):
    B, S, d = hidden_states.shape
    T = B * S
    x = hidden_states.reshape(T, d)

    out = pl.pallas_call(
        _moe_kernel,
        in_specs=[
            pl.BlockSpec(memory_space=pltpu.MemorySpace.VMEM),
            pl.BlockSpec(memory_space=pltpu.MemorySpace.VMEM),
            pl.BlockSpec(memory_space=pltpu.MemorySpace.VMEM),
            pl.BlockSpec(memory_space=pltpu.MemorySpace.HBM),
        ],
        out_specs=pl.BlockSpec(memory_space=pltpu.MemorySpace.VMEM),
        out_shape=jax.ShapeDtypeStruct((T, d), jnp.float32),
        scratch_shapes=[
            pltpu.VMEM((2, _NCH, _CH, d), jnp.float32),
            pltpu.SemaphoreType.DMA((2, _NCH)),
        ],
    )(x, gate_w, expert_b, expert_w)

    return out.reshape(B, S, d)
